# baseline (speedup 1.0000x reference)
import functools

import jax
import jax.numpy as jnp
from jax import lax
from jax.experimental import pallas as pl
from jax.experimental.pallas import tpu as pltpu
from jax.experimental.pallas import tpu_sc as plsc

_T = 16384
_D = 2048
_E = 16
_K = 2
_B = 512
_NB = _T // _B

_NC = 2
_NS = 16
_CH = _T // _NS
_EPW = _K * _CH
_ROW = 4 * _CH + _E


def _route_block(x_ref, w_ref, er_ref, sv_ref, cnt_ref, off_ref):
    pid = pl.program_id(0)

    @pl.when(pid == 0)
    def _():
        cnt_ref[...] = jnp.zeros_like(cnt_ref)

    scores = jax.nn.sigmoid(
        jnp.dot(x_ref[...], w_ref[...], preferred_element_type=jnp.float32))
    eii = lax.broadcasted_iota(jnp.int32, (_E, _E), 0)
    ejj = lax.broadcasted_iota(jnp.int32, (_E, _E), 1)
    triE = (eii < ejj).astype(jnp.float32)

    s1 = jnp.max(scores, axis=1, keepdims=True)
    m1 = (scores == s1).astype(jnp.float32)
    oh1f = jnp.where(jnp.dot(m1, triE, preferred_element_type=jnp.float32)
                     == 0.0, m1, 0.0)
    masked = scores - 2.0 * oh1f
    s2 = jnp.max(masked, axis=1, keepdims=True)
    m2 = (masked == s2).astype(jnp.float32)
    oh2f = jnp.where(jnp.dot(m2, triE, preferred_element_type=jnp.float32)
                     == 0.0, m2, 0.0)

    cf = oh1f + oh2f
    ii = lax.broadcasted_iota(jnp.int32, (_B, _B), 0)
    jj = lax.broadcasted_iota(jnp.int32, (_B, _B), 1)
    tri = (jj < ii).astype(jnp.float32)
    pref = jnp.dot(tri, cf, preferred_element_type=jnp.float32)
    carry = cnt_ref[...]
    g = pref + carry.astype(jnp.float32)
    gath = jnp.concatenate([oh1f, oh2f, oh1f * g, oh2f * g], axis=1)
    kk = lax.broadcasted_iota(jnp.int32, (4 * _E, 4), 0)
    ll = lax.broadcasted_iota(jnp.int32, (4 * _E, 4), 1)
    wsel = jnp.where(kk // _E == ll,
                     jnp.where(ll < 2, (kk % _E).astype(jnp.float32), 1.0),
                     0.0)
    red = jnp.dot(gath, wsel, preferred_element_type=jnp.float32,
                  precision=lax.Precision.HIGHEST).astype(jnp.int32)
    new_cnt = carry + jnp.sum(cf, axis=0, keepdims=True).astype(jnp.int32)
    cnt_ref[...] = new_cnt
    eii = lax.broadcasted_iota(jnp.int32, (_E, _E), 0)
    ejj = lax.broadcasted_iota(jnp.int32, (_E, _E), 1)
    off_ref[...] = jnp.dot(new_cnt.astype(jnp.float32),
                           (eii < ejj).astype(jnp.float32),
                           preferred_element_type=jnp.float32,
                           precision=lax.Precision.HIGHEST).astype(jnp.int32)

    er_ref[...] = red
    sv_ref[...] = jnp.concatenate([s1, s2], axis=1)


_route = pl.pallas_call(
    _route_block,
    grid=(_NB,),
    in_specs=[
        pl.BlockSpec((_B, _D), lambda i: (i, 0)),
        pl.BlockSpec((_D, _E), lambda i: (0, 0)),
    ],
    out_specs=[
        pl.BlockSpec((_B, 4), lambda i: (i, 0)),
        pl.BlockSpec((_B, 2), lambda i: (i, 0)),
        pl.BlockSpec((1, _E), lambda i: (0, 0)),
        pl.BlockSpec((1, _E), lambda i: (0, 0)),
    ],
    out_shape=[
        jax.ShapeDtypeStruct((_T, 4), jnp.int32),
        jax.ShapeDtypeStruct((_T, 2), jnp.float32),
        jax.ShapeDtypeStruct((1, _E), jnp.int32),
        jax.ShapeDtypeStruct((1, _E), jnp.int32),
    ],
)


@functools.cache
def _make_dispatch():
  return functools.partial(
    pl.kernel,
    out_type=(jax.ShapeDtypeStruct((_K * _T,), jnp.float32),
              jax.ShapeDtypeStruct((_K * _T,), jnp.int32)),
    mesh=plsc.VectorSubcoreMesh(core_axis_name="c", subcore_axis_name="s",
                                num_cores=_NC, num_subcores=_NS),
    compiler_params=pltpu.CompilerParams(needs_layout_passes=False),
    scratch_types=[
        pltpu.VMEM((_ROW,), jnp.int32),
        pltpu.VMEM((_EPW,), jnp.float32),
        pltpu.VMEM((_EPW,), jnp.int32),
        pltpu.VMEM((_EPW,), jnp.int32),
        pltpu.VMEM_SHARED((_K * _T,), jnp.float32),
        pltpu.VMEM_SHARED((_K * _T,), jnp.int32),
    ],
  )(_dispatch_body)


def _dispatch_body(er_hbm, sv_hbm, out_s_hbm, out_t_hbm,
                   er_v, sv_v, pos_v, tok_v, spm_s, spm_t):
    cid = lax.axis_index("c")
    sid = lax.axis_index("s")
    pltpu.sync_copy(er_hbm.at[sid], er_v)

    @pl.when(cid == 0)
    def _():
        pltpu.sync_copy(sv_hbm.at[sid], sv_v)

    base_tok = sid * _CH
    for j in range(_CH // 16):
        t_loc = j * 16 + lax.broadcasted_iota(jnp.int32, (16,), 0)
        e1j = plsc.load_gather(er_v, [4 * t_loc])
        e2j = plsc.load_gather(er_v, [4 * t_loc + 1])
        r1j = plsc.load_gather(er_v, [4 * t_loc + 2])
        r2j = plsc.load_gather(er_v, [4 * t_loc + 3])
        p1 = plsc.load_gather(er_v, [4 * _CH + e1j]) + r1j
        p2 = plsc.load_gather(er_v, [4 * _CH + e2j]) + r2j
        tok = base_tok + t_loc
        f1 = 2 * t_loc
        f2 = f1 + 1
        plsc.store_scatter(pos_v, [f1], p1)
        plsc.store_scatter(pos_v, [f2], p2)
        plsc.store_scatter(tok_v, [f1], tok)
        plsc.store_scatter(tok_v, [f2], tok)

    @pl.when(cid == 0)
    def _():
        pltpu.sync_copy(sv_v, spm_s.at[pos_v])

    @pl.when(cid == 1)
    def _():
        pltpu.sync_copy(tok_v, spm_t.at[pos_v])

    plsc.subcore_barrier()

    sl = pl.ds(sid * _EPW, _EPW)

    @pl.when(cid == 0)
    def _():
        pltpu.sync_copy(spm_s.at[sl], sv_v)
        pltpu.sync_copy(sv_v, out_s_hbm.at[sl])

    @pl.when(cid == 1)
    def _():
        pltpu.sync_copy(spm_t.at[sl], tok_v)
        pltpu.sync_copy(tok_v, out_t_hbm.at[sl])


def kernel(x, W_gate):
    er, sv, cnt, off = _route(x, W_gate)
    er_rows = jnp.concatenate(
        [er.reshape(_NS, 4 * _CH),
         jnp.broadcast_to(off.reshape(1, _E), (_NS, _E))], axis=1)
    sv_rows = sv.reshape(_NS, _EPW)
    out_s, out_t = _make_dispatch()(er_rows, sv_rows)
    return out_s, out_t, cnt.reshape(_E)


# speedup vs baseline: 1.0746x; 1.0746x over previous
import functools

import jax
import jax.numpy as jnp
from jax import lax
from jax.experimental import pallas as pl
from jax.experimental.pallas import tpu as pltpu
from jax.experimental.pallas import tpu_sc as plsc

_T = 16384
_D = 2048
_E = 16
_K = 2
_B = 512
_NB = _T // _B

_NC = 2
_NS = 16
_CH = _T // _NS
_EPW = _K * _CH
_ROW = 4 * _CH + _E


def _route_block(x_ref, w_ref, er_ref, sv_ref, cnt_ref, off_ref):
    pid = pl.program_id(0)

    @pl.when(pid == 0)
    def _():
        cnt_ref[...] = jnp.zeros_like(cnt_ref)

    scores = jax.nn.sigmoid(
        jnp.dot(x_ref[...], w_ref[...], preferred_element_type=jnp.float32))
    iota_e = lax.broadcasted_iota(jnp.int32, (_B, _E), 1)

    s1 = jnp.max(scores, axis=1, keepdims=True)
    e1 = jnp.min(jnp.where(scores == s1, iota_e, _E), axis=1, keepdims=True)
    oh1 = iota_e == e1
    masked = jnp.where(oh1, -1.0, scores)
    s2 = jnp.max(masked, axis=1, keepdims=True)
    e2 = jnp.min(jnp.where(masked == s2, iota_e, _E), axis=1, keepdims=True)
    oh2 = iota_e == e2

    c = oh1.astype(jnp.int32) + oh2.astype(jnp.int32)
    ii = lax.broadcasted_iota(jnp.int32, (_B, _B), 0)
    jj = lax.broadcasted_iota(jnp.int32, (_B, _B), 1)
    tri = (jj < ii).astype(jnp.float32)
    pref = jnp.dot(tri, c.astype(jnp.float32),
                   preferred_element_type=jnp.float32).astype(jnp.int32)
    carry = cnt_ref[...]
    g = pref + carry
    r1 = jnp.sum(jnp.where(oh1, g, 0), axis=1, keepdims=True)
    r2 = jnp.sum(jnp.where(oh2, g, 0), axis=1, keepdims=True)
    red = jnp.concatenate([e1, e2, r1, r2], axis=1)
    new_cnt = carry + jnp.sum(c, axis=0, keepdims=True)
    cnt_ref[...] = new_cnt
    eii = lax.broadcasted_iota(jnp.int32, (_E, _E), 0)
    ejj = lax.broadcasted_iota(jnp.int32, (_E, _E), 1)
    off_ref[...] = jnp.dot(new_cnt.astype(jnp.float32),
                           (eii < ejj).astype(jnp.float32),
                           preferred_element_type=jnp.float32,
                           precision=lax.Precision.HIGHEST).astype(jnp.int32)

    er_ref[...] = red
    sv_ref[...] = jnp.concatenate([s1, s2], axis=1)


_route = pl.pallas_call(
    _route_block,
    grid=(_NB,),
    in_specs=[
        pl.BlockSpec((_B, _D), lambda i: (i, 0)),
        pl.BlockSpec((_D, _E), lambda i: (0, 0)),
    ],
    out_specs=[
        pl.BlockSpec((_B, 4), lambda i: (i, 0)),
        pl.BlockSpec((_B, 2), lambda i: (i, 0)),
        pl.BlockSpec((1, _E), lambda i: (0, 0)),
        pl.BlockSpec((1, _E), lambda i: (0, 0)),
    ],
    out_shape=[
        jax.ShapeDtypeStruct((_T, 4), jnp.int32),
        jax.ShapeDtypeStruct((_T, 2), jnp.float32),
        jax.ShapeDtypeStruct((1, _E), jnp.int32),
        jax.ShapeDtypeStruct((1, _E), jnp.int32),
    ],
)


@functools.cache
def _make_dispatch():
  return functools.partial(
    pl.kernel,
    out_type=(jax.ShapeDtypeStruct((_K * _T,), jnp.float32),
              jax.ShapeDtypeStruct((_K * _T,), jnp.int32)),
    mesh=plsc.VectorSubcoreMesh(core_axis_name="c", subcore_axis_name="s",
                                num_cores=_NC, num_subcores=_NS),
    compiler_params=pltpu.CompilerParams(needs_layout_passes=False),
    scratch_types=[
        pltpu.VMEM((_ROW,), jnp.int32),
        pltpu.VMEM((_EPW,), jnp.float32),
        pltpu.VMEM((_EPW,), jnp.int32),
        pltpu.VMEM((_EPW,), jnp.int32),
        pltpu.VMEM_SHARED((_K * _T,), jnp.float32),
        pltpu.VMEM_SHARED((_K * _T,), jnp.int32),
    ],
  )(_dispatch_body)


def _dispatch_body(er_hbm, sv_hbm, out_s_hbm, out_t_hbm,
                   er_v, sv_v, pos_v, tok_v, spm_s, spm_t):
    cid = lax.axis_index("c")
    sid = lax.axis_index("s")
    pltpu.sync_copy(er_hbm.at[sid], er_v)

    @pl.when(cid == 0)
    def _():
        pltpu.sync_copy(sv_hbm.at[sid], sv_v)

    base_tok = sid * _CH
    for j in range(_CH // 16):
        t_loc = j * 16 + lax.broadcasted_iota(jnp.int32, (16,), 0)
        e1j = plsc.load_gather(er_v, [4 * t_loc])
        e2j = plsc.load_gather(er_v, [4 * t_loc + 1])
        r1j = plsc.load_gather(er_v, [4 * t_loc + 2])
        r2j = plsc.load_gather(er_v, [4 * t_loc + 3])
        p1 = plsc.load_gather(er_v, [4 * _CH + e1j]) + r1j
        p2 = plsc.load_gather(er_v, [4 * _CH + e2j]) + r2j
        tok = base_tok + t_loc
        f1 = 2 * t_loc
        f2 = f1 + 1
        plsc.store_scatter(pos_v, [f1], p1)
        plsc.store_scatter(pos_v, [f2], p2)
        plsc.store_scatter(tok_v, [f1], tok)
        plsc.store_scatter(tok_v, [f2], tok)

    @pl.when(cid == 0)
    def _():
        pltpu.sync_copy(sv_v, spm_s.at[pos_v])

    @pl.when(cid == 1)
    def _():
        pltpu.sync_copy(tok_v, spm_t.at[pos_v])

    plsc.subcore_barrier()

    sl = pl.ds(sid * _EPW, _EPW)

    @pl.when(cid == 0)
    def _():
        pltpu.sync_copy(spm_s.at[sl], sv_v)
        pltpu.sync_copy(sv_v, out_s_hbm.at[sl])

    @pl.when(cid == 1)
    def _():
        pltpu.sync_copy(spm_t.at[sl], tok_v)
        pltpu.sync_copy(tok_v, out_t_hbm.at[sl])


def kernel(x, W_gate):
    er, sv, cnt, off = _route(x, W_gate)
    er_rows = jnp.concatenate(
        [er.reshape(_NS, 4 * _CH),
         jnp.broadcast_to(off.reshape(1, _E), (_NS, _E))], axis=1)
    sv_rows = sv.reshape(_NS, _EPW)
    out_s, out_t = _make_dispatch()(er_rows, sv_rows)
    return out_s, out_t, cnt.reshape(_E)


# speedup vs baseline: 1.0968x; 1.0207x over previous
import functools

import jax
import jax.numpy as jnp
from jax import lax
from jax.experimental import pallas as pl
from jax.experimental.pallas import tpu as pltpu
from jax.experimental.pallas import tpu_sc as plsc

_T = 16384
_D = 2048
_E = 16
_K = 2
_B = 1024
_NB = _T // _B

_NC = 2
_NS = 16
_CH = _T // _NS
_EPW = _K * _CH
_ROW = 4 * _CH + _E


def _route_block(x_ref, w_ref, er_ref, sv_ref, cnt_ref, off_ref):
    pid = pl.program_id(0)

    @pl.when(pid == 0)
    def _():
        cnt_ref[...] = jnp.zeros_like(cnt_ref)

    scores = jax.nn.sigmoid(
        jnp.dot(x_ref[...], w_ref[...], preferred_element_type=jnp.float32))
    iota_e = lax.broadcasted_iota(jnp.int32, (_B, _E), 1)

    s1 = jnp.max(scores, axis=1, keepdims=True)
    e1 = jnp.min(jnp.where(scores == s1, iota_e, _E), axis=1, keepdims=True)
    oh1 = iota_e == e1
    masked = jnp.where(oh1, -1.0, scores)
    s2 = jnp.max(masked, axis=1, keepdims=True)
    e2 = jnp.min(jnp.where(masked == s2, iota_e, _E), axis=1, keepdims=True)
    oh2 = iota_e == e2

    c = oh1.astype(jnp.int32) + oh2.astype(jnp.int32)
    ii = lax.broadcasted_iota(jnp.int32, (_B, _B), 0)
    jj = lax.broadcasted_iota(jnp.int32, (_B, _B), 1)
    tri = (jj < ii).astype(jnp.float32)
    pref = jnp.dot(tri, c.astype(jnp.float32),
                   preferred_element_type=jnp.float32).astype(jnp.int32)
    carry = cnt_ref[...]
    g = pref + carry
    r1 = jnp.sum(jnp.where(oh1, g, 0), axis=1, keepdims=True)
    r2 = jnp.sum(jnp.where(oh2, g, 0), axis=1, keepdims=True)
    red = jnp.concatenate([e1, e2, r1, r2], axis=1)
    new_cnt = carry + jnp.sum(c, axis=0, keepdims=True)
    cnt_ref[...] = new_cnt
    eii = lax.broadcasted_iota(jnp.int32, (_E, _E), 0)
    ejj = lax.broadcasted_iota(jnp.int32, (_E, _E), 1)
    off_ref[...] = jnp.dot(new_cnt.astype(jnp.float32),
                           (eii < ejj).astype(jnp.float32),
                           preferred_element_type=jnp.float32,
                           precision=lax.Precision.HIGHEST).astype(jnp.int32)

    er_ref[...] = red
    sv_ref[...] = jnp.concatenate([s1, s2], axis=1)


_route = pl.pallas_call(
    _route_block,
    grid=(_NB,),
    in_specs=[
        pl.BlockSpec((_B, _D), lambda i: (i, 0)),
        pl.BlockSpec((_D, _E), lambda i: (0, 0)),
    ],
    out_specs=[
        pl.BlockSpec((_B, 4), lambda i: (i, 0)),
        pl.BlockSpec((_B, 2), lambda i: (i, 0)),
        pl.BlockSpec((1, _E), lambda i: (0, 0)),
        pl.BlockSpec((1, _E), lambda i: (0, 0)),
    ],
    out_shape=[
        jax.ShapeDtypeStruct((_T, 4), jnp.int32),
        jax.ShapeDtypeStruct((_T, 2), jnp.float32),
        jax.ShapeDtypeStruct((1, _E), jnp.int32),
        jax.ShapeDtypeStruct((1, _E), jnp.int32),
    ],
)


@functools.cache
def _make_dispatch():
  return functools.partial(
    pl.kernel,
    out_type=(jax.ShapeDtypeStruct((_K * _T,), jnp.float32),
              jax.ShapeDtypeStruct((_K * _T,), jnp.int32)),
    mesh=plsc.VectorSubcoreMesh(core_axis_name="c", subcore_axis_name="s",
                                num_cores=_NC, num_subcores=_NS),
    compiler_params=pltpu.CompilerParams(needs_layout_passes=False),
    scratch_types=[
        pltpu.VMEM((4 * _CH,), jnp.int32),
        pltpu.VMEM((_E,), jnp.int32),
        pltpu.VMEM((_EPW,), jnp.float32),
        pltpu.VMEM((_EPW,), jnp.int32),
        pltpu.VMEM((_EPW,), jnp.int32),
        pltpu.VMEM_SHARED((_K * _T,), jnp.float32),
        pltpu.VMEM_SHARED((_K * _T,), jnp.int32),
    ],
  )(_dispatch_body)


def _dispatch_body(er_hbm, off_hbm, sv_hbm, out_s_hbm, out_t_hbm,
                   er_v, off_v, sv_v, pos_v, tok_v, spm_s, spm_t):
    cid = lax.axis_index("c")
    sid = lax.axis_index("s")
    pltpu.sync_copy(er_hbm.at[pl.ds(sid * 4 * _CH, 4 * _CH)], er_v)
    pltpu.sync_copy(off_hbm, off_v)

    @pl.when(cid == 0)
    def _():
        pltpu.sync_copy(sv_hbm.at[pl.ds(sid * _EPW, _EPW)], sv_v)

    base_tok = sid * _CH
    for j in range(_CH // 16):
        t_loc = j * 16 + lax.broadcasted_iota(jnp.int32, (16,), 0)
        e1j = plsc.load_gather(er_v, [4 * t_loc])
        e2j = plsc.load_gather(er_v, [4 * t_loc + 1])
        r1j = plsc.load_gather(er_v, [4 * t_loc + 2])
        r2j = plsc.load_gather(er_v, [4 * t_loc + 3])
        p1 = plsc.load_gather(off_v, [e1j]) + r1j
        p2 = plsc.load_gather(off_v, [e2j]) + r2j
        tok = base_tok + t_loc
        f1 = 2 * t_loc
        f2 = f1 + 1
        plsc.store_scatter(pos_v, [f1], p1)
        plsc.store_scatter(pos_v, [f2], p2)
        plsc.store_scatter(tok_v, [f1], tok)
        plsc.store_scatter(tok_v, [f2], tok)

    @pl.when(cid == 0)
    def _():
        pltpu.sync_copy(sv_v, spm_s.at[pos_v])

    @pl.when(cid == 1)
    def _():
        pltpu.sync_copy(tok_v, spm_t.at[pos_v])

    plsc.subcore_barrier()

    sl = pl.ds(sid * _EPW, _EPW)

    @pl.when(cid == 0)
    def _():
        pltpu.sync_copy(spm_s.at[sl], sv_v)
        pltpu.sync_copy(sv_v, out_s_hbm.at[sl])

    @pl.when(cid == 1)
    def _():
        pltpu.sync_copy(spm_t.at[sl], tok_v)
        pltpu.sync_copy(tok_v, out_t_hbm.at[sl])


def kernel(x, W_gate):
    er, sv, cnt, off = _route(x, W_gate)
    out_s, out_t = _make_dispatch()(er.reshape(-1), off.reshape(-1),
                                    sv.reshape(-1))
    return out_s, out_t, cnt.reshape(_E)
